# Initial kernel scaffold; baseline (speedup 1.0000x reference)
import functools
import jax
import jax.numpy as jnp
from jax import lax
from jax.experimental import pallas as pl
from jax.experimental.pallas import tpu as pltpu
from jax.experimental.pallas import tpu_sc as plsc

NC, NS = 2, 16
NW = NC * NS
D = 50
B = 4096 * 200
CHUNK = 128
GROUP = 4
GROW = CHUNK * GROUP
ROWS_PER_W = B // NW
NGROUPS = ROWS_PER_W // GROW

_mesh = plsc.VectorSubcoreMesh(core_axis_name="c", subcore_axis_name="s")


@functools.partial(
    pl.kernel,
    out_type=jax.ShapeDtypeStruct((B, D), jnp.float32),
    mesh=_mesh,
    scratch_types=[
        pltpu.VMEM((GROUP, CHUNK), jnp.int32),
        pltpu.VMEM((GROW, D), jnp.float32),
        pltpu.SemaphoreType.DMA,
    ],
)
def _emb_lookup(ids_hbm, table_hbm, out_hbm, idx_v, rows_v, gsem):
    wid = lax.axis_index("s") * NC + lax.axis_index("c")
    row_base = wid * ROWS_PER_W

    def body(g, carry):
        pltpu.sync_copy(ids_hbm.at[wid, g], idx_v)
        copies = []
        for j in range(GROUP):
            copies.append(
                pltpu.async_copy(
                    table_hbm.at[idx_v.at[j]],
                    rows_v.at[pl.ds(j * CHUNK, CHUNK)],
                    gsem,
                )
            )
        for c in copies:
            c.wait()
        pltpu.sync_copy(rows_v, out_hbm.at[pl.ds(row_base + g * GROW, GROW)])
        return carry

    lax.fori_loop(0, NGROUPS, body, 0)


def kernel(pos_ids, table):
    ids = pos_ids.reshape(NW, NGROUPS, GROUP, CHUNK)
    out = _emb_lookup(ids, table)
    return out.reshape(pos_ids.shape[0], pos_ids.shape[1], D)


# speedup vs baseline: 3.5727x; 3.5727x over previous
import functools
import jax
import jax.numpy as jnp
from jax import lax
from jax.experimental import pallas as pl
from jax.experimental.pallas import tpu as pltpu
from jax.experimental.pallas import tpu_sc as plsc

NC, NS = 2, 16
NW = NC * NS
D = 50
DP = 64
B = 4096 * 200
CHUNK = 128
GROUP = 4
GROW = CHUNK * GROUP
ROWS_PER_W = B // NW
NGROUPS = ROWS_PER_W // GROW

_mesh = plsc.VectorSubcoreMesh(core_axis_name="c", subcore_axis_name="s")


@functools.partial(
    pl.kernel,
    out_type=jax.ShapeDtypeStruct((B, DP), jnp.float32),
    mesh=_mesh,
    scratch_types=[
        pltpu.VMEM((GROUP, CHUNK), jnp.int32),
        pltpu.VMEM((GROW, DP), jnp.float32),
        pltpu.SemaphoreType.DMA,
    ],
    compiler_params=pltpu.CompilerParams(use_tc_tiling_on_sc=False),
)
def _emb_lookup(ids_hbm, table_hbm, out_hbm, idx_v, rows_v, gsem):
    wid = lax.axis_index("s") * NC + lax.axis_index("c")
    row_base = wid * ROWS_PER_W

    def body(g, carry):
        pltpu.sync_copy(ids_hbm.at[wid, g], idx_v)
        copies = []
        for j in range(GROUP):
            copies.append(
                pltpu.async_copy(
                    table_hbm.at[idx_v.at[j]],
                    rows_v.at[pl.ds(j * CHUNK, CHUNK)],
                    gsem,
                )
            )
        for c in copies:
            c.wait()
        pltpu.sync_copy(rows_v, out_hbm.at[pl.ds(row_base + g * GROW, GROW)])
        return carry

    lax.fori_loop(0, NGROUPS, body, 0)


def kernel(pos_ids, table):
    ids = pos_ids.reshape(NW, NGROUPS, GROUP, CHUNK)
    table_p = jnp.pad(table, ((0, 0), (0, DP - D)))
    out = _emb_lookup(ids, table_p)
    return out[:, :D].reshape(pos_ids.shape[0], pos_ids.shape[1], D)
